# baseline (speedup 1.0000x reference)
import functools

import jax
import jax.numpy as jnp
from jax import lax
from jax.experimental import pallas as pl
from jax.experimental.pallas import tpu as pltpu, tpu_sc as plsc

NC = 2
NS = 16
NW = NC * NS
B = 16384
D = 32
BPW = B // NW
NGRP = BPW // 16


def _sc_gather_reduce(user, item, neg, ibias, ulat8, ilat8):
    mesh = plsc.VectorSubcoreMesh(core_axis_name="c", subcore_axis_name="s")

    @functools.partial(
        pl.kernel,
        out_type=[
            jax.ShapeDtypeStruct((B,), jnp.float32),
            jax.ShapeDtypeStruct((B,), jnp.float32),
            jax.ShapeDtypeStruct((B,), jnp.float32),
            jax.ShapeDtypeStruct((NW * 16,), jnp.float32),
        ],
        mesh=mesh,
        compiler_params=pltpu.CompilerParams(needs_layout_passes=False),
        scratch_types=[
            pltpu.VMEM((BPW,), jnp.int32),
            pltpu.VMEM((BPW,), jnp.int32),
            pltpu.VMEM((BPW,), jnp.int32),
            pltpu.VMEM((BPW * D,), jnp.float32),
            pltpu.VMEM((BPW * D,), jnp.float32),
            pltpu.VMEM((BPW * D,), jnp.float32),
            pltpu.VMEM((BPW,), jnp.float32),
            pltpu.VMEM((BPW,), jnp.float32),
            pltpu.VMEM((BPW,), jnp.float32),
            pltpu.VMEM((BPW,), jnp.float32),
            pltpu.VMEM((BPW,), jnp.float32),
            pltpu.VMEM((16,), jnp.float32),
            pltpu.SemaphoreType.DMA,
            pltpu.SemaphoreType.DMA,
        ],
    )
    def k(user_h, item_h, neg_h, ibias_h, ulat_h, ilat_h,
          score_h, usq_h, isq_h, nsq_h,
          uflat, iflat, nflat, ue_f, ie_f, nie_f, ib_v, nib_v,
          score_v, usq_v, isq_v, nsq_v, semr, semb):
        wid = lax.axis_index("s") * NC + lax.axis_index("c")
        base = wid * BPW

        pltpu.sync_copy(user_h.at[pl.ds(base, BPW)], uflat)
        pltpu.sync_copy(item_h.at[pl.ds(base, BPW)], iflat)
        pltpu.sync_copy(neg_h.at[pl.ds(base, BPW)], nflat)

        bias_copies = []
        for j in range(4):
            sl = pl.ds(j * 128, 128)
            bias_copies.append(
                pltpu.async_copy(ibias_h.at[iflat.at[sl]], ib_v.at[sl], semb))
            bias_copies.append(
                pltpu.async_copy(ibias_h.at[nflat.at[sl]], nib_v.at[sl], semb))

        def row_body(g, _):
            uvec = uflat[pl.ds(g * 16, 16)]
            ivec = iflat[pl.ds(g * 16, 16)]
            nvec = nflat[pl.ds(g * 16, 16)]
            for j in range(16):
                sl = pl.ds((g * 16 + j) * D, D)
                u = uvec[j]
                pltpu.async_copy(ulat_h.at[pl.ds(u * D, D)], ue_f.at[sl], semr)
                it = ivec[j]
                pltpu.async_copy(ilat_h.at[pl.ds(it * D, D)], ie_f.at[sl], semr)
                n = nvec[j]
                pltpu.async_copy(ilat_h.at[pl.ds(n * D, D)], nie_f.at[sl], semr)
            return 0

        lax.fori_loop(0, NGRP, row_body, 0)

        for buf in (ue_f, ie_f, nie_f):
            pltpu.make_async_copy(score_h, buf, semr).wait()
        for c in bias_copies:
            c.wait()

        iota16 = lax.iota(jnp.int32, 16)

        def g_body(g, nacc):
            goff = g * 16
            flat0 = (goff + iota16) * D
            s = ib_v[pl.ds(goff, 16)] - nib_v[pl.ds(goff, 16)]
            u = jnp.zeros((16,), jnp.float32)
            i2 = jnp.zeros((16,), jnp.float32)
            for d in range(D):
                ue = plsc.load_gather(ue_f, [flat0 + d])
                ie = plsc.load_gather(ie_f, [flat0 + d])
                nie = plsc.load_gather(nie_f, [flat0 + d])
                s = s + ue * (ie - nie)
                u = u + ue * ue
                i2 = i2 + ie * ie
                nacc = nacc + nie * nie
            score_v[pl.ds(goff, 16)] = s
            usq_v[pl.ds(goff, 16)] = u
            isq_v[pl.ds(goff, 16)] = i2
            return nacc

        nacc = lax.fori_loop(0, NGRP, g_body, jnp.zeros((16,), jnp.float32))
        nsq_v[...] = nacc

        pltpu.sync_copy(score_v, score_h.at[pl.ds(base, BPW)])
        pltpu.sync_copy(usq_v, usq_h.at[pl.ds(base, BPW)])
        pltpu.sync_copy(isq_v, isq_h.at[pl.ds(base, BPW)])
        pltpu.sync_copy(nsq_v, nsq_h.at[pl.ds(wid * 16, 16)])

    return k(user, item, neg, ibias, ulat8, ilat8)


def _tc_finish(score, usq, isq, nsq):
    def body(score_ref, usq_ref, isq_ref, nsq_ref, bpr_ref, l2_ref):
        s = score_ref[...]
        softplus = jnp.maximum(-s, 0.0) + jnp.log1p(jnp.exp(-jnp.abs(s)))
        bpr_ref[0, 0] = jnp.sum(softplus)
        l2_ref[0, 0] = (jnp.sum(jnp.sqrt(usq_ref[...]))
                        + jnp.sum(jnp.sqrt(isq_ref[...]))
                        + jnp.sqrt(jnp.sum(nsq_ref[...])))

    return pl.pallas_call(
        body,
        out_shape=[jax.ShapeDtypeStruct((1, 1), jnp.float32)] * 2,
        out_specs=[pl.BlockSpec(memory_space=pltpu.SMEM)] * 2,
    )(score, usq, isq, nsq)


def kernel(user, item, neg_item, user_bais, item_bais, user_laten, item_laten):
    ulat_f = user_laten.reshape(-1)
    ilat_f = item_laten.reshape(-1)
    score, usq, isq, nsq = _sc_gather_reduce(
        user, item, neg_item, item_bais.reshape(-1), ulat_f, ilat_f)
    bpr, l2 = _tc_finish(score.reshape(128, 128), usq.reshape(128, 128),
                         isq.reshape(128, 128), nsq.reshape(4, 128))
    return (bpr[0, 0], l2[0, 0])
